# baseline (speedup 1.0000x reference)
import functools

import jax
import jax.numpy as jnp
from jax import lax
from jax.experimental import pallas as pl
from jax.experimental.pallas import tpu as pltpu
from jax.experimental.pallas import tpu_sc as plsc

_DIM = 64
_BATCH = 16384
_ROWS = 1000001
_TCOLS_FULL = _ROWS // 128
_TAIL_BASE = _TCOLS_FULL * 128
_TCOLS = _TCOLS_FULL + 1

_info = plsc.get_sparse_core_info()
_NC, _NS = _info.num_cores, _info.num_subcores
_NW = _NC * _NS
_SH = -(-_TCOLS // _NW)
_WTC = 4
_WLANES = _WTC * 128
_BTC = 2 * _WTC
_NBUCK = -(-_SH // _BTC)
_SENT = 255 << 21
_TRASH = _BATCH
_STG = 96
_L0CAP = _BATCH + 16
_L1CAP = _BATCH + 16 * (_NBUCK + 1)

_mesh = plsc.VectorSubcoreMesh(core_axis_name="c", subcore_axis_name="s")


def _append(ref, base, x, mask):
    pos = plsc.cumsum(mask.astype(jnp.int32)) - 1
    idx = jnp.where(mask, base + pos, 0)
    plsc.store_scatter(ref, [idx], x, mask=mask)


@functools.partial(
    pl.kernel,
    mesh=_mesh,
    out_type=jax.ShapeDtypeStruct((_BATCH + 128, 128), jnp.float32),
    scratch_types=[
        pltpu.VMEM((_BATCH,), jnp.int32),
        pltpu.VMEM((_L0CAP,), jnp.int32),
        pltpu.VMEM((_L1CAP,), jnp.int32),
        pltpu.VMEM((2, _DIM, _WLANES), jnp.float32),
        pltpu.VMEM((_STG, 128), jnp.float32),
        pltpu.VMEM((_STG,), jnp.int32),
        pltpu.SMEM((_NBUCK + 1,), jnp.int32),
        pltpu.SemaphoreType.DMA,
        pltpu.SemaphoreType.DMA,
    ],
    compiler_params=pltpu.CompilerParams(needs_layout_passes=False),
)
def _gather_kernel(table_t, labels_hbm, tail_t, out_hbm,
                   labv, l0, l1, win, staging, jidx, off_s, sem, osem):
    wid = lax.axis_index("s") * _NC + lax.axis_index("c")
    ncols = jnp.minimum(_SH, _TCOLS - wid * _SH)
    owns_tail = (_TCOLS_FULL - wid * _SH >= 0) & (_TCOLS_FULL - wid * _SH < ncols)
    nfull = ncols - owns_tail.astype(jnp.int32)
    nwin_main = lax.div(nfull + _WTC - 1, _WTC)
    nwin = nwin_main + owns_tail.astype(jnp.int32)
    iota = lax.iota(jnp.int32, 16)

    def win_cbase(w):
        return jnp.minimum(wid * _SH + w * _WTC, _TCOLS_FULL - _WTC)

    def start_win(w, buf):
        is_tail = owns_tail & (w == nwin_main)
        col = pl.multiple_of(jnp.where(is_tail, 0, win_cbase(w)) * 128, 128)

        @pl.when(jnp.logical_not(is_tail))
        def _():
            for o in range(_DIM // 8):
                pltpu.async_copy(
                    table_t.at[pl.ds(8 * o, 8), pl.ds(col, _WLANES)],
                    win.at[buf, pl.ds(8 * o, 8), :],
                    sem,
                )

        @pl.when(is_tail)
        def _():
            pltpu.async_copy(tail_t, win.at[buf, :, pl.ds(0, 128)], sem)

    def wait_win(w, buf):
        is_tail = owns_tail & (w == nwin_main)

        @pl.when(jnp.logical_not(is_tail))
        def _():
            for o in range(_DIM // 8):
                pltpu.make_async_copy(
                    table_t.at[pl.ds(0, 8), pl.ds(0, _WLANES)],
                    win.at[buf, pl.ds(8 * o, 8), :],
                    sem,
                ).wait()

        @pl.when(is_tail)
        def _():
            pltpu.make_async_copy(
                tail_t, win.at[buf, :, pl.ds(0, 128)], sem
            ).wait()

    def flush():
        pltpu.async_copy(staging, out_hbm.at[jidx], osem).wait()
        reset_jidx()

    start_win(jnp.int32(0), jnp.int32(0))

    @pl.when(nwin > 1)
    def _():
        start_win(jnp.int32(1), jnp.int32(1))

    pltpu.sync_copy(labels_hbm, labv)

    def p1(v, c0):
        r = labv[pl.ds(v * 16, 16)]
        ltc = lax.shift_right_logical(r, 7) - wid * _SH
        mask = (ltc >= 0) & (ltc < ncols)
        word = (
            lax.shift_left(ltc, 21)
            | lax.shift_left(r & 127, 14)
            | (v * 16 + iota)
        )
        _append(l0, c0, word, mask)
        return c0 + jnp.sum(mask.astype(jnp.int32))

    c0 = lax.fori_loop(0, _BATCH // 16, p1, jnp.int32(0))
    l0[pl.ds(c0, 16)] = jnp.full((16,), _SENT, jnp.int32)
    nv0 = lax.shift_right_logical(c0 + 15, 4)

    def p2(b, c1):
        off_s[b] = c1

        def scan(v, c):
            word = l0[pl.ds(v * 16, 16)]
            mask = lax.shift_right_logical(word, 24) == b
            _append(l1, c, word, mask)
            return c + jnp.sum(mask.astype(jnp.int32))

        c1 = lax.fori_loop(0, nv0, scan, c1)
        l1[pl.ds(c1, 16)] = jnp.full((16,), _SENT, jnp.int32)
        c1 = (c1 + 15) & ~jnp.int32(15)
        return c1

    c1 = lax.fori_loop(0, _NBUCK, p2, jnp.int32(0))
    off_s[_NBUCK] = c1

    def reset_jidx():
        for t in range(_STG // 16):
            jidx[pl.ds(t * 16, 16)] = _TRASH + t * 16 + iota

    reset_jidx()

    def per_window(w, m):
        buf = lax.rem(w, 2)
        wait_win(w, buf)
        is_tail = owns_tail & (w == nwin_main)
        cbase = jnp.where(is_tail, _TCOLS_FULL, win_cbase(w))
        b = jnp.where(
            is_tail,
            lax.shift_right_logical(nfull, 3),
            jnp.minimum(lax.shift_right_logical(w, 1), _NBUCK - 1),
        )
        vlo = lax.shift_right_logical(off_s[b], 4)
        vhi = lax.shift_right_logical(off_s[b + 1], 4)
        wref = win.at[buf]

        def per_vreg(v, m):
            word = l1[pl.ds(v * 16, 16)]
            ltc = lax.shift_right_logical(word, 21)
            gtc = ltc + wid * _SH
            in_tail = gtc == _TCOLS_FULL
            mask = (
                jnp.where(
                    is_tail,
                    in_tail,
                    (ltc >= w * _WTC) & (ltc < (w + 1) * _WTC)
                    & jnp.logical_not(in_tail),
                )
                & (ltc < 255)
            )
            valid = jnp.sum(mask.astype(jnp.int32))

            @pl.when(valid > 0)
            def _():
                lane = lax.shift_right_logical(word, 14) & 127
                lp = jnp.where(mask, (gtc - cbase) * 128 + lane, 0)
                mi = mask.astype(jnp.int32)
                mv = jnp.where(mask, m + plsc.cumsum(mi) - 1, 0)
                _append(jidx, m, word & 16383, mask)
                for q in range(16):
                    @pl.when(mi[q] > 0)
                    def _():
                        lq = jnp.full((16,), lp[q], jnp.int32)
                        row = staging.at[mv[q]]
                        for k in range(_DIM // 16):
                            vals = plsc.load_gather(
                                wref, [iota + 16 * k, lq]
                            )
                            row[pl.ds(16 * k, 16)] = vals

            m2 = m + valid

            @pl.when(m2 > _STG - 16)
            def _():
                flush()

            return jnp.where(m2 > _STG - 16, jnp.int32(0), m2)

        m = lax.fori_loop(vlo, vhi, per_vreg, m)

        @pl.when(w + 2 < nwin)
        def _():
            start_win(w + 2, buf)

        return m

    m = lax.fori_loop(0, nwin, per_window, jnp.int32(0))

    @pl.when(m > 0)
    def _():
        flush()


def kernel(labels, embedding_table):
    table_t = embedding_table.T
    tail_t = jnp.pad(
        table_t[:, _TAIL_BASE:], ((0, 0), (0, 128 - (_ROWS - _TAIL_BASE)))
    )
    out = _gather_kernel(table_t, labels.astype(jnp.int32), tail_t)
    return out[:_BATCH, :_DIM]


# speedup vs baseline: 1.0024x; 1.0024x over previous
import functools

import jax
import jax.numpy as jnp
from jax import lax
from jax.experimental import pallas as pl
from jax.experimental.pallas import tpu as pltpu
from jax.experimental.pallas import tpu_sc as plsc

_DIM = 64
_BATCH = 16384
_ROWS = 1000001
_TCOLS_FULL = _ROWS // 128
_TAIL_BASE = _TCOLS_FULL * 128
_TCOLS = _TCOLS_FULL + 1

_info = plsc.get_sparse_core_info()
_NC, _NS = _info.num_cores, _info.num_subcores
_NW = _NC * _NS
_SH = -(-_TCOLS // _NW)
_WTC = 4
_WLANES = _WTC * 128
_BTC = 2 * _WTC
_NBUCK = -(-_SH // _BTC)
_SENT = 255 << 21
_TRASH = _BATCH
_STG = 96
_L0CAP = _BATCH + 16
_L1CAP = _BATCH + 16 * (_NBUCK + 1)

_mesh = plsc.VectorSubcoreMesh(core_axis_name="c", subcore_axis_name="s")


def _append(ref, base, x, mask):
    pos = plsc.cumsum(mask.astype(jnp.int32)) - 1
    idx = jnp.where(mask, base + pos, 0)
    plsc.store_scatter(ref, [idx], x, mask=mask)


@functools.partial(
    pl.kernel,
    mesh=_mesh,
    out_type=jax.ShapeDtypeStruct((_BATCH + 128, 128), jnp.float32),
    scratch_types=[
        pltpu.VMEM((_BATCH,), jnp.int32),
        pltpu.VMEM((_L0CAP,), jnp.int32),
        pltpu.VMEM((_L1CAP,), jnp.int32),
        pltpu.VMEM((2, _DIM, _WLANES), jnp.float32),
        pltpu.VMEM((_STG, 128), jnp.float32),
        pltpu.VMEM((_STG,), jnp.int32),
        pltpu.SMEM((_NBUCK + 1,), jnp.int32),
        pltpu.SemaphoreType.DMA,
        pltpu.SemaphoreType.DMA,
    ],
    compiler_params=pltpu.CompilerParams(needs_layout_passes=False),
)
def _gather_kernel(table_t, labels_hbm, tail_t, out_hbm,
                   labv, l0, l1, win, staging, jidx, off_s, sem, osem):
    wid = lax.axis_index("s") * _NC + lax.axis_index("c")
    ncols = jnp.minimum(_SH, _TCOLS - wid * _SH)
    owns_tail = (_TCOLS_FULL - wid * _SH >= 0) & (_TCOLS_FULL - wid * _SH < ncols)
    nfull = ncols - owns_tail.astype(jnp.int32)
    nwin_main = lax.div(nfull + _WTC - 1, _WTC)
    nwin = nwin_main + owns_tail.astype(jnp.int32)
    iota = lax.iota(jnp.int32, 16)

    def win_cbase(w):
        return jnp.minimum(wid * _SH + w * _WTC, _TCOLS_FULL - _WTC)

    def start_win(w, buf):
        is_tail = owns_tail & (w == nwin_main)
        col = pl.multiple_of(jnp.where(is_tail, 0, win_cbase(w)) * 128, 128)

        @pl.when(jnp.logical_not(is_tail))
        def _():
            pltpu.async_copy(
                table_t.at[:, pl.ds(col, _WLANES)], win.at[buf], sem
            )

        @pl.when(is_tail)
        def _():
            pltpu.async_copy(tail_t, win.at[buf, :, pl.ds(0, 128)], sem)

    def wait_win(w, buf):
        is_tail = owns_tail & (w == nwin_main)

        @pl.when(jnp.logical_not(is_tail))
        def _():
            pltpu.make_async_copy(
                table_t.at[:, pl.ds(0, _WLANES)], win.at[buf], sem
            ).wait()

        @pl.when(is_tail)
        def _():
            pltpu.make_async_copy(
                tail_t, win.at[buf, :, pl.ds(0, 128)], sem
            ).wait()

    def flush():
        pltpu.async_copy(staging, out_hbm.at[jidx], osem).wait()
        reset_jidx()

    start_win(jnp.int32(0), jnp.int32(0))

    @pl.when(nwin > 1)
    def _():
        start_win(jnp.int32(1), jnp.int32(1))

    pltpu.sync_copy(labels_hbm, labv)

    def p1(v, c0):
        r = labv[pl.ds(v * 16, 16)]
        ltc = lax.shift_right_logical(r, 7) - wid * _SH
        mask = (ltc >= 0) & (ltc < ncols)
        word = (
            lax.shift_left(ltc, 21)
            | lax.shift_left(r & 127, 14)
            | (v * 16 + iota)
        )
        _append(l0, c0, word, mask)
        return c0 + jnp.sum(mask.astype(jnp.int32))

    c0 = lax.fori_loop(0, _BATCH // 16, p1, jnp.int32(0))
    l0[pl.ds(c0, 16)] = jnp.full((16,), _SENT, jnp.int32)
    nv0 = lax.shift_right_logical(c0 + 15, 4)

    def p2(b, c1):
        off_s[b] = c1

        def scan(v, c):
            word = l0[pl.ds(v * 16, 16)]
            mask = lax.shift_right_logical(word, 24) == b
            _append(l1, c, word, mask)
            return c + jnp.sum(mask.astype(jnp.int32))

        c1 = lax.fori_loop(0, nv0, scan, c1)
        l1[pl.ds(c1, 16)] = jnp.full((16,), _SENT, jnp.int32)
        c1 = (c1 + 15) & ~jnp.int32(15)
        return c1

    c1 = lax.fori_loop(0, _NBUCK, p2, jnp.int32(0))
    off_s[_NBUCK] = c1

    def reset_jidx():
        for t in range(_STG // 16):
            jidx[pl.ds(t * 16, 16)] = _TRASH + t * 16 + iota

    reset_jidx()

    def per_window(w, m):
        buf = lax.rem(w, 2)
        wait_win(w, buf)
        is_tail = owns_tail & (w == nwin_main)
        cbase = jnp.where(is_tail, _TCOLS_FULL, win_cbase(w))
        b = jnp.where(
            is_tail,
            lax.shift_right_logical(nfull, 3),
            jnp.minimum(lax.shift_right_logical(w, 1), _NBUCK - 1),
        )
        vlo = lax.shift_right_logical(off_s[b], 4)
        vhi = lax.shift_right_logical(off_s[b + 1], 4)
        wref = win.at[buf]

        def per_vreg(v, m):
            word = l1[pl.ds(v * 16, 16)]
            ltc = lax.shift_right_logical(word, 21)
            gtc = ltc + wid * _SH
            in_tail = gtc == _TCOLS_FULL
            mask = (
                jnp.where(
                    is_tail,
                    in_tail,
                    (ltc >= w * _WTC) & (ltc < (w + 1) * _WTC)
                    & jnp.logical_not(in_tail),
                )
                & (ltc < 255)
            )
            valid = jnp.sum(mask.astype(jnp.int32))

            @pl.when(valid > 0)
            def _():
                lane = lax.shift_right_logical(word, 14) & 127
                lp = jnp.where(mask, (gtc - cbase) * 128 + lane, 0)
                mi = mask.astype(jnp.int32)
                mv = jnp.where(mask, m + plsc.cumsum(mi) - 1, 0)
                _append(jidx, m, word & 16383, mask)
                for q in range(16):
                    @pl.when(mi[q] > 0)
                    def _():
                        lq = jnp.full((16,), lp[q], jnp.int32)
                        row = staging.at[mv[q]]
                        for k in range(_DIM // 16):
                            vals = plsc.load_gather(
                                wref, [iota + 16 * k, lq]
                            )
                            row[pl.ds(16 * k, 16)] = vals

            m2 = m + valid

            @pl.when(m2 > _STG - 16)
            def _():
                flush()

            return jnp.where(m2 > _STG - 16, jnp.int32(0), m2)

        m = lax.fori_loop(vlo, vhi, per_vreg, m)

        @pl.when(w + 2 < nwin)
        def _():
            start_win(w + 2, buf)

        return m

    m = lax.fori_loop(0, nwin, per_window, jnp.int32(0))

    @pl.when(m > 0)
    def _():
        flush()


def kernel(labels, embedding_table):
    table_t = embedding_table.T
    tail_t = jnp.pad(
        table_t[:, _TAIL_BASE:], ((0, 0), (0, 128 - (_ROWS - _TAIL_BASE)))
    )
    out = _gather_kernel(table_t, labels.astype(jnp.int32), tail_t)
    return out[:_BATCH, :_DIM]
